# baseline (speedup 1.0000x reference)
import jax
import jax.numpy as jnp
from jax.experimental import pallas as pl
from jax.experimental.pallas import tpu as pltpu

DIM_CODES = 64
DICT_SIZE = 8192
DIM_EMBED = 32
BATCH = 128
C_GRP = 8
N_GRP = DIM_CODES // C_GRP
K_Q = DICT_SIZE // 4
N_BUF = 5


def _fused_body(xt_ref, d_ref, idx_ref, ce_ref, oh_hbm, ring_ref, kio_ref, sem):
    c = pl.program_id(0)
    g = c // C_GRP
    s = jax.lax.rem(c, C_GRP)

    @pl.when(c == 0)
    def _init_iota():
        kio_ref[...] = jax.lax.broadcasted_iota(
            jnp.int32, (DICT_SIZE, BATCH), 0)

    @pl.when(s == 0)
    def _claim_bufs():
        for q in range(4):
            qg = 4 * g + q

            @pl.when(qg >= N_BUF)
            def _wait_one():
                buf = jax.lax.rem(qg, N_BUF)
                pltpu.make_async_copy(
                    ring_ref.at[buf],
                    oh_hbm.at[:, pl.ds(g * C_GRP, C_GRP), pl.ds(q * K_Q, K_Q)],
                    sem.at[buf]).wait()

    xt = xt_ref[0]
    dc = d_ref[0]
    xyT = jax.lax.dot_general(dc, xt, (((1,), (0,)), ((), ())),
                              preferred_element_type=jnp.float32)
    y_sq = jnp.sum(dc * dc, axis=1, keepdims=True)
    x_sq = jnp.sum(xt * xt, axis=0, keepdims=True)
    distT = x_sq - 2.0 * xyT + y_sq
    m = jnp.min(distT, axis=0, keepdims=True)
    kio = kio_ref[...]
    cand = jnp.where(distT == m, kio, DICT_SIZE)
    idxv = jnp.min(cand, axis=0, keepdims=True)
    idx_ref[0] = idxv
    onehotT = (kio == idxv).astype(jnp.float32)
    ceT = jax.lax.dot_general(dc, onehotT, (((0,), (0,)), ((), ())),
                              preferred_element_type=jnp.float32)
    ce_ref[0] = ceT

    idx_col = jnp.transpose(idxv)
    for q in range(4):
        qg = 4 * g + q
        buf = jax.lax.rem(qg, N_BUF)
        kio2 = jax.lax.broadcasted_iota(
            jnp.int32, (BATCH, K_Q), 1) + (q * K_Q)
        ring_ref[buf, :, s, :] = (kio2 == idx_col).astype(jnp.float32)

    @pl.when(s == C_GRP - 1)
    def _send_group():
        for q in range(4):
            buf = jax.lax.rem(4 * g + q, N_BUF)
            pltpu.make_async_copy(
                ring_ref.at[buf],
                oh_hbm.at[:, pl.ds(g * C_GRP, C_GRP), pl.ds(q * K_Q, K_Q)],
                sem.at[buf]).start()

    @pl.when(c == DIM_CODES - 1)
    def _drain():
        for qg in range(4 * N_GRP - N_BUF, 4 * N_GRP):
            buf = qg % N_BUF
            gq, q = qg // 4, qg % 4
            pltpu.make_async_copy(
                ring_ref.at[buf],
                oh_hbm.at[:, pl.ds(gq * C_GRP, C_GRP), pl.ds(q * K_Q, K_Q)],
                sem.at[buf]).wait()


def kernel(x, dictionary):
    xt = x.reshape(BATCH, DIM_CODES, DIM_EMBED).transpose(1, 2, 0)

    idx_t, ce_t, one_hot = pl.pallas_call(
        _fused_body,
        grid=(DIM_CODES,),
        in_specs=[
            pl.BlockSpec((1, DIM_EMBED, BATCH), lambda c: (c, 0, 0)),
            pl.BlockSpec((1, DICT_SIZE, DIM_EMBED), lambda c: (c, 0, 0)),
        ],
        out_specs=[
            pl.BlockSpec((1, 1, BATCH), lambda c: (c, 0, 0)),
            pl.BlockSpec((1, DIM_EMBED, BATCH), lambda c: (c, 0, 0)),
            pl.BlockSpec(memory_space=pltpu.MemorySpace.HBM),
        ],
        out_shape=[
            jax.ShapeDtypeStruct((DIM_CODES, 1, BATCH), jnp.int32),
            jax.ShapeDtypeStruct((DIM_CODES, DIM_EMBED, BATCH), jnp.float32),
            jax.ShapeDtypeStruct((BATCH, DIM_CODES, DICT_SIZE), jnp.float32),
        ],
        scratch_shapes=[
            pltpu.VMEM((N_BUF, BATCH, C_GRP, K_Q), jnp.float32),
            pltpu.VMEM((DICT_SIZE, BATCH), jnp.int32),
            pltpu.SemaphoreType.DMA((N_BUF,)),
        ],
        compiler_params=pltpu.CompilerParams(
            vmem_limit_bytes=63 * 1024 * 1024,
        ),
    )(xt, dictionary)

    cw_e = ce_t.transpose(2, 0, 1).reshape(BATCH, DIM_CODES * DIM_EMBED)
    return cw_e, cw_e, one_hot


# speedup vs baseline: 1.4199x; 1.4199x over previous
import jax
import jax.numpy as jnp
from jax.experimental import pallas as pl
from jax.experimental.pallas import tpu as pltpu

DIM_CODES = 64
DICT_SIZE = 8192
DIM_EMBED = 32
BATCH = 128
N_SLOT = 4


def _fused_body(xt_ref, d_ref, idx_ref, ce_ref, oh_hbm, oh_ref, kio_ref, sem):
    c = pl.program_id(0)
    slot = jax.lax.rem(c, N_SLOT)

    @pl.when(c == 0)
    def _init_iota():
        kio_ref[...] = jax.lax.broadcasted_iota(
            jnp.int32, (DICT_SIZE, BATCH), 0)

    @pl.when(c >= N_SLOT)
    def _wait_prev():
        pltpu.make_async_copy(oh_ref.at[slot], oh_hbm.at[:, c - N_SLOT, :],
                              sem.at[slot]).wait()

    xt = xt_ref[0]
    dc = d_ref[0]
    xyT = jax.lax.dot_general(dc, xt, (((1,), (0,)), ((), ())),
                              preferred_element_type=jnp.float32)
    y_sq = jnp.sum(dc * dc, axis=1, keepdims=True)
    x_sq = jnp.sum(xt * xt, axis=0, keepdims=True)
    distT = x_sq - 2.0 * xyT + y_sq
    m = jnp.min(distT, axis=0, keepdims=True)
    kio = kio_ref[...]
    cand = jnp.where(distT == m, kio, DICT_SIZE)
    idxv = jnp.min(cand, axis=0, keepdims=True)
    idx_ref[0] = idxv
    onehotT = (kio == idxv).astype(jnp.float32)
    ceT = jax.lax.dot_general(dc, onehotT, (((0,), (0,)), ((), ())),
                              preferred_element_type=jnp.float32)
    ce_ref[0] = ceT

    idx_col = jnp.transpose(idxv)
    kio2 = jax.lax.broadcasted_iota(jnp.int32, (BATCH, DICT_SIZE), 1)
    oh_ref[slot] = (kio2 == idx_col).astype(jnp.float32)
    pltpu.make_async_copy(oh_ref.at[slot], oh_hbm.at[:, c, :],
                          sem.at[slot]).start()

    @pl.when(c == DIM_CODES - 1)
    def _drain():
        for j in range(N_SLOT):
            cc = DIM_CODES - N_SLOT + j
            pltpu.make_async_copy(oh_ref.at[cc % N_SLOT], oh_hbm.at[:, cc, :],
                                  sem.at[cc % N_SLOT]).wait()


def kernel(x, dictionary):
    xt = x.reshape(BATCH, DIM_CODES, DIM_EMBED).transpose(1, 2, 0)

    idx_t, ce_t, one_hot = pl.pallas_call(
        _fused_body,
        grid=(DIM_CODES,),
        in_specs=[
            pl.BlockSpec((1, DIM_EMBED, BATCH), lambda c: (c, 0, 0)),
            pl.BlockSpec((1, DICT_SIZE, DIM_EMBED), lambda c: (c, 0, 0)),
        ],
        out_specs=[
            pl.BlockSpec((1, 1, BATCH), lambda c: (c, 0, 0)),
            pl.BlockSpec((1, DIM_EMBED, BATCH), lambda c: (c, 0, 0)),
            pl.BlockSpec(memory_space=pltpu.MemorySpace.HBM),
        ],
        out_shape=[
            jax.ShapeDtypeStruct((DIM_CODES, 1, BATCH), jnp.int32),
            jax.ShapeDtypeStruct((DIM_CODES, DIM_EMBED, BATCH), jnp.float32),
            jax.ShapeDtypeStruct((BATCH, DIM_CODES, DICT_SIZE), jnp.float32),
        ],
        scratch_shapes=[
            pltpu.VMEM((N_SLOT, BATCH, DICT_SIZE), jnp.float32),
            pltpu.VMEM((DICT_SIZE, BATCH), jnp.int32),
            pltpu.SemaphoreType.DMA((N_SLOT,)),
        ],
    )(xt, dictionary)

    cw_e = ce_t.transpose(2, 0, 1).reshape(BATCH, DIM_CODES * DIM_EMBED)
    return cw_e, cw_e, one_hot
